# baseline (speedup 1.0000x reference)
import jax
import jax.numpy as jnp
from jax.experimental import pallas as pl
import functools

M, N, P, Q = 16, 192, 56, 56
BLK = 4


def _body(ar, br, cr, ai, bi, ci, o_r, o_i):
    ra = ar[...]
    ia = ai[...]
    rb = br[...]
    ib = bi[...]
    rc = cr[...]
    ic = ci[...]
    ma = ra * ra + ia * ia
    mb = rb * rb + ib * ib
    mc = rc * rc + ic * ic
    b_wins = mb > ma
    r1 = jnp.where(b_wins, rb, ra)
    i1 = jnp.where(b_wins, ib, ia)
    m1 = jnp.maximum(ma, mb)
    c_wins = mc > m1
    o_r[...] = jnp.where(c_wins, rc, r1)
    o_i[...] = jnp.where(c_wins, ic, i1)


@jax.jit
def _tc_max_fusion(ar, br, cr, ai, bi, ci):
    spec = pl.BlockSpec((M, BLK, P, Q), lambda i: (0, i, 0, 0))
    return pl.pallas_call(
        _body,
        grid=(N // BLK,),
        in_specs=[spec] * 6,
        out_specs=[spec] * 2,
        out_shape=[jax.ShapeDtypeStruct((M, N, P, Q), jnp.float32)] * 2,
    )(ar, br, cr, ai, bi, ci)


def kernel(Fea_A_r, Fea_B_r, Fea_C_r, Fea_A_i, Fea_B_i, Fea_C_i):
    return _tc_max_fusion(Fea_A_r, Fea_B_r, Fea_C_r, Fea_A_i, Fea_B_i, Fea_C_i)


# speedup vs baseline: 1.0009x; 1.0009x over previous
import jax
import jax.numpy as jnp
from jax.experimental import pallas as pl
import functools

M, N, P, Q = 16, 192, 56, 56
BLK = 96


def _body(ar, br, cr, ai, bi, ci, o_r, o_i):
    ra = ar[...]
    ia = ai[...]
    rb = br[...]
    ib = bi[...]
    rc = cr[...]
    ic = ci[...]
    ma = ra * ra + ia * ia
    mb = rb * rb + ib * ib
    mc = rc * rc + ic * ic
    b_wins = mb > ma
    r1 = jnp.where(b_wins, rb, ra)
    i1 = jnp.where(b_wins, ib, ia)
    m1 = jnp.maximum(ma, mb)
    c_wins = mc > m1
    o_r[...] = jnp.where(c_wins, rc, r1)
    o_i[...] = jnp.where(c_wins, ic, i1)


@jax.jit
def _tc_max_fusion(ar, br, cr, ai, bi, ci):
    spec = pl.BlockSpec((1, BLK, P, Q), lambda i, j: (i, j, 0, 0))
    return pl.pallas_call(
        _body,
        grid=(M, N // BLK),
        in_specs=[spec] * 6,
        out_specs=[spec] * 2,
        out_shape=[jax.ShapeDtypeStruct((M, N, P, Q), jnp.float32)] * 2,
    )(ar, br, cr, ai, bi, ci)


def kernel(Fea_A_r, Fea_B_r, Fea_C_r, Fea_A_i, Fea_B_i, Fea_C_i):
    return _tc_max_fusion(Fea_A_r, Fea_B_r, Fea_C_r, Fea_A_i, Fea_B_i, Fea_C_i)


# speedup vs baseline: 6.3830x; 6.3775x over previous
import jax
import jax.numpy as jnp
from jax.experimental import pallas as pl
import functools

M, N, P, Q = 16, 192, 56, 56
R = M * P * Q
BR = 1568


def _body(ar, br, cr, ai, bi, ci, o_r, o_i):
    ra = ar[...]
    ia = ai[...]
    rb = br[...]
    ib = bi[...]
    rc = cr[...]
    ic = ci[...]
    ma = ra * ra + ia * ia
    mb = rb * rb + ib * ib
    mc = rc * rc + ic * ic
    b_wins = mb > ma
    r1 = jnp.where(b_wins, rb, ra)
    i1 = jnp.where(b_wins, ib, ia)
    m1 = jnp.maximum(ma, mb)
    c_wins = mc > m1
    o_r[...] = jnp.where(c_wins, rc, r1)
    o_i[...] = jnp.where(c_wins, ic, i1)


@jax.jit
def _tc_max_fusion(ar, br, cr, ai, bi, ci):
    spec = pl.BlockSpec((BR, N), lambda i: (i, 0))
    return pl.pallas_call(
        _body,
        grid=(R // BR,),
        in_specs=[spec] * 6,
        out_specs=[spec] * 2,
        out_shape=[jax.ShapeDtypeStruct((R, N), jnp.float32)] * 2,
    )(ar, br, cr, ai, bi, ci)


def kernel(Fea_A_r, Fea_B_r, Fea_C_r, Fea_A_i, Fea_B_i, Fea_C_i):
    t = lambda x: x.transpose(0, 2, 3, 1).reshape(R, N)
    out_r, out_i = _tc_max_fusion(
        t(Fea_A_r), t(Fea_B_r), t(Fea_C_r),
        t(Fea_A_i), t(Fea_B_i), t(Fea_C_i),
    )
    u = lambda x: x.reshape(M, P, Q, N).transpose(0, 3, 1, 2)
    return u(out_r), u(out_i)
